# baseline (device time: 15356 ns/iter reference)
import jax
import jax.numpy as jnp
from jax import lax
from jax.experimental import pallas as pl
from jax.experimental.pallas import tpu as pltpu

N_DEV = 4
E_PER = 2
COMM_DTYPE = jnp.float8_e4m3fn
W_SCALE = 32.0


def kernel(x, router_W, route_idx, expert_W, shared_W):
    m, d = x.shape
    e_per, _, h = expert_W.shape
    n_exp = router_W.shape[1]

    def body(x_hbm, router_W_ref, route_idx_ref, ew_hbm, sw_hbm,
             out_hbm, x_s, ew_s, sw_s, out_s, loc_bf, from_l, from_r, opp,
             dma_sems, send_sems, recv_sems):
        my_pos = lax.axis_index("i")
        left = lax.rem(my_pos - 1 + N_DEV, N_DEV)
        right = lax.rem(my_pos + 1, N_DEV)

        barrier_sem = pltpu.get_barrier_semaphore()
        for nbr in (left, right):
            pl.semaphore_signal(
                barrier_sem, inc=1,
                device_id=(nbr,), device_id_type=pl.DeviceIdType.MESH,
            )

        ew_dma = pltpu.make_async_copy(ew_hbm, ew_s, dma_sems.at[0])
        x_dma = pltpu.make_async_copy(x_hbm, x_s, dma_sems.at[1])
        sw_dma = pltpu.make_async_copy(sw_hbm, sw_s, dma_sems.at[2])
        ew_dma.start()
        x_dma.start()
        sw_dma.start()

        ew_dma.wait()
        loc_bf[...] = (ew_s[...] * W_SCALE).astype(COMM_DTYPE)

        pl.semaphore_wait(barrier_sem, 2)

        def copy(src, dst, sem, target):
            return pltpu.make_async_remote_copy(
                src_ref=src, dst_ref=dst,
                send_sem=send_sems.at[sem], recv_sem=recv_sems.at[sem],
                device_id=(target,), device_id_type=pl.DeviceIdType.MESH,
            )

        p1_r0 = copy(loc_bf.at[0], from_l.at[0], 0, right)
        p1_l1 = copy(loc_bf.at[1], from_r.at[1], 1, left)
        p1_r1 = copy(loc_bf.at[1], from_l.at[1], 2, right)
        p1_l0 = copy(loc_bf.at[0], from_r.at[0], 3, left)
        p1_r0.start()
        p1_l1.start()
        p1_r1.start()
        p1_l0.start()

        x_dma.wait()
        xv = x_s[...]
        scores = jnp.dot(xv, router_W_ref[...],
                         preferred_element_type=jnp.float32)
        s_max = jnp.max(scores, axis=-1, keepdims=True)
        p = jnp.exp(scores - s_max)
        probs = p / jnp.sum(p, axis=-1, keepdims=True)
        route = route_idx_ref[...]
        col = lax.broadcasted_iota(jnp.int32, (m, n_exp), 1)
        gate = jnp.sum(jnp.where(col == route, probs, 0.0), axis=1)

        xb = xv.astype(jnp.bfloat16)

        def contrib(eid, w2d):
            w = jnp.where(route[:, 0] == eid, gate, 0.0)
            if w2d.dtype == COMM_DTYPE:
                w = w * (1.0 / W_SCALE)
            return jnp.dot(w.astype(jnp.bfloat16)[:, None] * xb,
                           w2d.astype(jnp.bfloat16),
                           preferred_element_type=jnp.float32)

        sw_dma.wait()
        out = jnp.dot(xb, sw_s[...].astype(jnp.bfloat16),
                      preferred_element_type=jnp.float32)
        out += contrib(my_pos * E_PER + 0, ew_s[0])
        out += contrib(my_pos * E_PER + 1, ew_s[1])

        p1_r0.wait_recv()
        p2_r = copy(from_l.at[0], opp.at[0], 4, right)
        p2_r.start()
        p1_l1.wait_recv()
        p2_l = copy(from_r.at[1], opp.at[1], 5, left)
        p2_l.start()

        out += contrib(left * E_PER + 0, from_l[0])
        out += contrib(right * E_PER + 1, from_r[1])

        p1_r1.wait_recv()
        out += contrib(left * E_PER + 1, from_l[1])
        p1_l0.wait_recv()
        out += contrib(right * E_PER + 0, from_r[0])

        opposite = lax.rem(my_pos + 2, N_DEV)
        p2_r.wait_recv()
        out += contrib(opposite * E_PER + 0, opp[0])
        p2_l.wait_recv()
        out += contrib(opposite * E_PER + 1, opp[1])

        out_s[...] = out
        out_dma = pltpu.make_async_copy(out_s, out_hbm, dma_sems.at[3])
        out_dma.start()
        for rdma in (p1_r0, p1_l1, p1_r1, p1_l0, p2_r, p2_l):
            rdma.wait_send()
        out_dma.wait()

    return pl.pallas_call(
        body,
        out_shape=jax.ShapeDtypeStruct((m, h), jnp.float32),
        in_specs=[
            pl.BlockSpec(memory_space=pl.ANY),
            pl.BlockSpec(memory_space=pltpu.VMEM),
            pl.BlockSpec(memory_space=pltpu.VMEM),
            pl.BlockSpec(memory_space=pl.ANY),
            pl.BlockSpec(memory_space=pl.ANY),
        ],
        out_specs=pl.BlockSpec(memory_space=pl.ANY),
        scratch_shapes=[
            pltpu.VMEM((m, d), jnp.float32),
            pltpu.VMEM((e_per, d, h), jnp.float32),
            pltpu.VMEM((d, h), jnp.float32),
            pltpu.VMEM((m, h), jnp.float32),
            pltpu.VMEM((e_per, d, h), COMM_DTYPE),
            pltpu.VMEM((e_per, d, h), COMM_DTYPE),
            pltpu.VMEM((e_per, d, h), COMM_DTYPE),
            pltpu.VMEM((e_per, d, h), COMM_DTYPE),
            pltpu.SemaphoreType.DMA((4,)),
            pltpu.SemaphoreType.DMA((6,)),
            pltpu.SemaphoreType.DMA((6,)),
        ],
        compiler_params=pltpu.CompilerParams(collective_id=0),
    )(x, router_W, route_idx, expert_W, shared_W)


# device time: 15285 ns/iter; 1.0046x vs baseline; 1.0046x over previous
import jax
import jax.numpy as jnp
from jax import lax
from jax.experimental import pallas as pl
from jax.experimental.pallas import tpu as pltpu

N_DEV = 4
E_PER = 2
COMM_DTYPE = jnp.float8_e4m3fn
W_SCALE = 32.0


def kernel(x, router_W, route_idx, expert_W, shared_W):
    m, d = x.shape
    e_per, _, h = expert_W.shape
    n_exp = router_W.shape[1]

    def body(x_hbm, router_W_ref, route_idx_ref, ew_hbm, sw_hbm,
             out_hbm, x_s, ew_s, sw_s, out_s, loc_bf, from_l, from_r, opp,
             dma_sems, send_sems, recv_sems):
        my_pos = lax.axis_index("i")
        left = lax.rem(my_pos - 1 + N_DEV, N_DEV)
        right = lax.rem(my_pos + 1, N_DEV)

        barrier_sem = pltpu.get_barrier_semaphore()
        for nbr in (left, right):
            pl.semaphore_signal(
                barrier_sem, inc=1,
                device_id=(nbr,), device_id_type=pl.DeviceIdType.MESH,
            )

        ew_dma = pltpu.make_async_copy(ew_hbm, ew_s, dma_sems.at[0])
        x_dma = pltpu.make_async_copy(x_hbm, x_s, dma_sems.at[1])
        sw_dma = pltpu.make_async_copy(sw_hbm, sw_s, dma_sems.at[2])
        ew_dma.start()
        x_dma.start()
        sw_dma.start()

        ew_dma.wait()
        loc_bf[...] = (ew_s[...] * W_SCALE).astype(COMM_DTYPE)

        pl.semaphore_wait(barrier_sem, 2)

        def copy(src, dst, sem, target):
            return pltpu.make_async_remote_copy(
                src_ref=src, dst_ref=dst,
                send_sem=send_sems.at[sem], recv_sem=recv_sems.at[sem],
                device_id=(target,), device_id_type=pl.DeviceIdType.MESH,
            )

        p1_r0 = copy(loc_bf.at[0], from_l.at[0], 0, right)
        p1_l1 = copy(loc_bf.at[1], from_r.at[1], 1, left)
        p1_r1 = copy(loc_bf.at[1], from_l.at[1], 2, right)
        p1_l0 = copy(loc_bf.at[0], from_r.at[0], 3, left)
        p1_r0.start()
        p1_l1.start()
        p1_r1.start()
        p1_l0.start()

        x_dma.wait()
        xv = x_s[...]
        scores = jnp.dot(xv, router_W_ref[...],
                         preferred_element_type=jnp.float32)
        s_max = jnp.max(scores, axis=-1, keepdims=True)
        p = jnp.exp(scores - s_max)
        probs = p / jnp.sum(p, axis=-1, keepdims=True)
        route = route_idx_ref[...]
        col = lax.broadcasted_iota(jnp.int32, (m, n_exp), 1)
        gate = jnp.sum(jnp.where(col == route, probs, 0.0), axis=1)

        xb = xv.astype(jnp.bfloat16)

        def contrib(eid, w2d):
            w = jnp.where(route[:, 0] == eid, gate, 0.0)
            if w2d.dtype == COMM_DTYPE:
                w = w * (1.0 / W_SCALE)
            return jnp.dot(w.astype(jnp.bfloat16)[:, None] * xb,
                           w2d.astype(jnp.bfloat16),
                           preferred_element_type=jnp.float32)

        sw_dma.wait()
        out = jnp.dot(xb, sw_s[...].astype(jnp.bfloat16),
                      preferred_element_type=jnp.float32)
        out += contrib(my_pos * E_PER + 0, ew_s[0])
        out += contrib(my_pos * E_PER + 1, ew_s[1])

        p1_r0.wait_recv()
        p2_r = copy(from_l.at[0], opp.at[0], 4, right)
        p2_r.start()
        p1_l1.wait_recv()
        p2_l = copy(from_r.at[1], opp.at[1], 5, left)
        p2_l.start()

        out += contrib(left * E_PER + 0, from_l[0])
        out += contrib(right * E_PER + 1, from_r[1])

        p1_r1.wait_recv()
        out += contrib(left * E_PER + 1, from_l[1])
        p1_l0.wait_recv()
        out += contrib(right * E_PER + 0, from_r[0])

        opposite = lax.rem(my_pos + 2, N_DEV)
        p2_r.wait_recv()
        out += contrib(opposite * E_PER + 0, opp[0])
        p2_l.wait_recv()
        out += contrib(opposite * E_PER + 1, opp[1])

        out_s[...] = out
        out_dma = pltpu.make_async_copy(out_s, out_hbm, dma_sems.at[3])
        out_dma.start()
        for rdma in (p1_r0, p1_l1, p1_r1, p1_l0, p2_r, p2_l):
            rdma.wait_send()
        out_dma.wait()

    return pl.pallas_call(
        body,
        out_shape=jax.ShapeDtypeStruct((m, h), jnp.float32),
        in_specs=[
            pl.BlockSpec(memory_space=pltpu.MemorySpace.HBM),
            pl.BlockSpec(memory_space=pltpu.VMEM),
            pl.BlockSpec(memory_space=pltpu.VMEM),
            pl.BlockSpec(memory_space=pltpu.MemorySpace.HBM),
            pl.BlockSpec(memory_space=pltpu.MemorySpace.HBM),
        ],
        out_specs=pl.BlockSpec(memory_space=pltpu.MemorySpace.HBM),
        scratch_shapes=[
            pltpu.VMEM((m, d), jnp.float32),
            pltpu.VMEM((e_per, d, h), jnp.float32),
            pltpu.VMEM((d, h), jnp.float32),
            pltpu.VMEM((m, h), jnp.float32),
            pltpu.VMEM((e_per, d, h), COMM_DTYPE),
            pltpu.VMEM((e_per, d, h), COMM_DTYPE),
            pltpu.VMEM((e_per, d, h), COMM_DTYPE),
            pltpu.VMEM((e_per, d, h), COMM_DTYPE),
            pltpu.SemaphoreType.DMA((4,)),
            pltpu.SemaphoreType.DMA((6,)),
            pltpu.SemaphoreType.DMA((6,)),
        ],
        compiler_params=pltpu.CompilerParams(collective_id=0),
    )(x, router_W, route_idx, expert_W, shared_W)


# device time: 14443 ns/iter; 1.0632x vs baseline; 1.0583x over previous
import jax
import jax.numpy as jnp
from jax import lax
from jax.experimental import pallas as pl
from jax.experimental.pallas import tpu as pltpu

N_DEV = 4
E_PER = 2
COMM_DTYPE = jnp.float8_e4m3fn
W_SCALE = 32.0


def kernel(x, router_W, route_idx, expert_W, shared_W):
    m, d = x.shape
    e_per, _, h = expert_W.shape
    n_exp = router_W.shape[1]

    scores = x @ router_W
    probs = jax.nn.softmax(scores, axis=-1)
    col = lax.broadcasted_iota(jnp.int32, (m, n_exp), 1)
    gate8 = jnp.where(col == route_idx, probs, 0.0)

    def body(x_ref, gate8_ref, expert_W_ref, shared_W_ref,
             out_ref, loc_q, from_l, from_r, opp, send_sems, recv_sems):
        my_pos = lax.axis_index("i")
        left = lax.rem(my_pos - 1 + N_DEV, N_DEV)
        right = lax.rem(my_pos + 1, N_DEV)

        loc_q[...] = (expert_W_ref[...] * W_SCALE).astype(COMM_DTYPE)

        barrier_sem = pltpu.get_barrier_semaphore()
        for nbr in (left, right):
            pl.semaphore_signal(
                barrier_sem, inc=1,
                device_id=(nbr,), device_id_type=pl.DeviceIdType.MESH,
            )
        pl.semaphore_wait(barrier_sem, 2)

        def copy(src, dst, sem, target):
            return pltpu.make_async_remote_copy(
                src_ref=src, dst_ref=dst,
                send_sem=send_sems.at[sem], recv_sem=recv_sems.at[sem],
                device_id=(target,), device_id_type=pl.DeviceIdType.MESH,
            )

        p1_r0 = copy(loc_q.at[0], from_l.at[0], 0, right)
        p1_l1 = copy(loc_q.at[1], from_r.at[1], 1, left)
        p1_r1 = copy(loc_q.at[1], from_l.at[1], 2, right)
        p1_l0 = copy(loc_q.at[0], from_r.at[0], 3, left)
        p1_r0.start()
        p1_l1.start()
        p1_r1.start()
        p1_l0.start()

        xb = x_ref[...].astype(jnp.bfloat16)
        gate8v = gate8_ref[...]
        ecol = lax.broadcasted_iota(jnp.int32, (m, n_exp), 1)

        def contrib(eid, w2d, scale):
            w = jnp.sum(jnp.where(ecol == eid, gate8v, 0.0), axis=1) * scale
            return jnp.dot(w.astype(jnp.bfloat16)[:, None] * xb,
                           w2d.astype(jnp.bfloat16),
                           preferred_element_type=jnp.float32)

        out = jnp.dot(xb, shared_W_ref[...].astype(jnp.bfloat16),
                      preferred_element_type=jnp.float32)
        out += contrib(my_pos * E_PER + 0, expert_W_ref[0], 1.0)
        out += contrib(my_pos * E_PER + 1, expert_W_ref[1], 1.0)

        p1_r0.wait_recv()
        p2_r = copy(from_l.at[0], opp.at[0], 4, right)
        p2_r.start()
        p1_l1.wait_recv()
        p2_l = copy(from_r.at[1], opp.at[1], 5, left)
        p2_l.start()

        inv = 1.0 / W_SCALE
        out += contrib(left * E_PER + 0, from_l[0], inv)
        out += contrib(right * E_PER + 1, from_r[1], inv)

        p1_r1.wait_recv()
        out += contrib(left * E_PER + 1, from_l[1], inv)
        p1_l0.wait_recv()
        out += contrib(right * E_PER + 0, from_r[0], inv)

        opposite = lax.rem(my_pos + 2, N_DEV)
        p2_r.wait_recv()
        out += contrib(opposite * E_PER + 0, opp[0], inv)
        p2_l.wait_recv()
        out += contrib(opposite * E_PER + 1, opp[1], inv)

        for rdma in (p1_r0, p1_l1, p1_r1, p1_l0, p2_r, p2_l):
            rdma.wait_send()
        out_ref[...] = out

    return pl.pallas_call(
        body,
        out_shape=jax.ShapeDtypeStruct((m, h), jnp.float32),
        in_specs=[pl.BlockSpec(memory_space=pltpu.VMEM)] * 4,
        out_specs=pl.BlockSpec(memory_space=pltpu.VMEM),
        scratch_shapes=[
            pltpu.VMEM((e_per, d, h), COMM_DTYPE),
            pltpu.VMEM((e_per, d, h), COMM_DTYPE),
            pltpu.VMEM((e_per, d, h), COMM_DTYPE),
            pltpu.VMEM((e_per, d, h), COMM_DTYPE),
            pltpu.SemaphoreType.DMA((6,)),
            pltpu.SemaphoreType.DMA((6,)),
        ],
        compiler_params=pltpu.CompilerParams(collective_id=0),
    )(x, gate8, expert_W, shared_W)


# device time: 14358 ns/iter; 1.0695x vs baseline; 1.0059x over previous
import jax
import jax.numpy as jnp
from jax import lax
from jax.experimental import pallas as pl
from jax.experimental.pallas import tpu as pltpu

N_DEV = 4
E_PER = 2
COMM_DTYPE = jnp.float8_e4m3fn
W_SCALE = 32.0


def kernel(x, router_W, route_idx, expert_W, shared_W):
    m, d = x.shape
    e_per, _, h = expert_W.shape
    n_exp = router_W.shape[1]

    scores = x @ router_W
    probs = jax.nn.softmax(scores, axis=-1)
    col = lax.broadcasted_iota(jnp.int32, (m, n_exp), 1)
    gate8 = jnp.where(col == route_idx, probs, 0.0)

    def body(x_ref, gate8_ref, expert_W_ref, shared_W_ref,
             out_ref, loc_q, from_l, from_r, opp, send_sems, recv_sems):
        my_pos = lax.axis_index("i")
        left = lax.rem(my_pos - 1 + N_DEV, N_DEV)
        right = lax.rem(my_pos + 1, N_DEV)

        barrier_sem = pltpu.get_barrier_semaphore()
        for nbr in (left, right):
            pl.semaphore_signal(
                barrier_sem, inc=1,
                device_id=(nbr,), device_id_type=pl.DeviceIdType.MESH,
            )

        loc_q[...] = (expert_W_ref[...] * W_SCALE).astype(COMM_DTYPE)

        pl.semaphore_wait(barrier_sem, 2)

        def copy(src, dst, sem, target):
            return pltpu.make_async_remote_copy(
                src_ref=src, dst_ref=dst,
                send_sem=send_sems.at[sem], recv_sem=recv_sems.at[sem],
                device_id=(target,), device_id_type=pl.DeviceIdType.MESH,
            )

        p1_r0 = copy(loc_q.at[0], from_l.at[0], 0, right)
        p1_l1 = copy(loc_q.at[1], from_r.at[1], 1, left)
        p1_r1 = copy(loc_q.at[1], from_l.at[1], 2, right)
        p1_l0 = copy(loc_q.at[0], from_r.at[0], 3, left)
        p1_r0.start()
        p1_l1.start()
        p1_r1.start()
        p1_l0.start()

        xb = x_ref[...].astype(jnp.bfloat16)
        gate8v = gate8_ref[...]
        ecol = lax.broadcasted_iota(jnp.int32, (m, n_exp), 1)

        def contrib(eid, w2d, scale):
            w = jnp.sum(jnp.where(ecol == eid, gate8v, 0.0), axis=1) * scale
            return jnp.dot(w.astype(jnp.bfloat16)[:, None] * xb,
                           w2d.astype(jnp.bfloat16),
                           preferred_element_type=jnp.float32)

        out = jnp.dot(xb, shared_W_ref[...].astype(jnp.bfloat16),
                      preferred_element_type=jnp.float32)
        out += contrib(my_pos * E_PER + 0, expert_W_ref[0], 1.0)
        out += contrib(my_pos * E_PER + 1, expert_W_ref[1], 1.0)

        p1_r0.wait_recv()
        p2_r = copy(from_l.at[0], opp.at[0], 4, right)
        p2_r.start()
        p1_l1.wait_recv()
        p2_l = copy(from_r.at[1], opp.at[1], 5, left)
        p2_l.start()

        inv = 1.0 / W_SCALE
        out += contrib(left * E_PER + 0, from_l[0], inv)
        out += contrib(right * E_PER + 1, from_r[1], inv)

        p1_r1.wait_recv()
        out += contrib(left * E_PER + 1, from_l[1], inv)
        p1_l0.wait_recv()
        out += contrib(right * E_PER + 0, from_r[0], inv)

        opposite = lax.rem(my_pos + 2, N_DEV)
        p2_r.wait_recv()
        out += contrib(opposite * E_PER + 0, opp[0], inv)
        p2_l.wait_recv()
        out += contrib(opposite * E_PER + 1, opp[1], inv)

        for rdma in (p1_r0, p1_l1, p1_r1, p1_l0, p2_r, p2_l):
            rdma.wait_send()
        out_ref[...] = out

    return pl.pallas_call(
        body,
        out_shape=jax.ShapeDtypeStruct((m, h), jnp.float32),
        in_specs=[pl.BlockSpec(memory_space=pltpu.VMEM)] * 4,
        out_specs=pl.BlockSpec(memory_space=pltpu.VMEM),
        scratch_shapes=[
            pltpu.VMEM((e_per, d, h), COMM_DTYPE),
            pltpu.VMEM((e_per, d, h), COMM_DTYPE),
            pltpu.VMEM((e_per, d, h), COMM_DTYPE),
            pltpu.VMEM((e_per, d, h), COMM_DTYPE),
            pltpu.SemaphoreType.DMA((6,)),
            pltpu.SemaphoreType.DMA((6,)),
        ],
        compiler_params=pltpu.CompilerParams(collective_id=0),
    )(x, gate8, expert_W, shared_W)
